# baseline (speedup 1.0000x reference)
import functools

import jax
import jax.numpy as jnp
import numpy as np
from jax import lax
from jax.experimental import pallas as pl
from jax.experimental.pallas import tpu as pltpu
from jax.experimental.pallas import tpu_sc as plsc

_NUM_CORES = 2
_NUM_SUBCORES = 16
_LANES = 16
_NW = _NUM_CORES * _NUM_SUBCORES

_CHUNK = 128


def _make_emb_kernel(n_rows, seq_len, dim):
    rows_per_w = n_rows // _NW
    chunks_per_w = rows_per_w // _CHUNK
    vecs_per_row = dim // _LANES
    mesh = plsc.VectorSubcoreMesh(
        core_axis_name="c", subcore_axis_name="s",
        num_cores=_NUM_CORES, num_subcores=_NUM_SUBCORES)

    @functools.partial(
        pl.kernel,
        out_type=jax.ShapeDtypeStruct((n_rows, dim), jnp.float32),
        mesh=mesh,
        scratch_types=[
            pltpu.VMEM((_CHUNK,), jnp.int32),
            pltpu.VMEM((_CHUNK, dim), jnp.float32),
            pltpu.VMEM((2 * seq_len, dim), jnp.float32),
            pltpu.SemaphoreType.DMA,
        ],
        compiler_params=pltpu.CompilerParams(use_tc_tiling_on_sc=False),
    )
    def emb(ids_hbm, table_hbm, pe_hbm, out_hbm, idx_v, rows_v, pe_v, sem):
        wid = lax.axis_index("c") * _NUM_SUBCORES + lax.axis_index("s")
        pltpu.sync_copy(pe_hbm, pe_v)
        wbase = wid * rows_per_w

        def chunk_body(c, carry):
            base = wbase + c * _CHUNK
            p = lax.rem(c * _CHUNK, seq_len)
            pltpu.sync_copy(ids_hbm.at[pl.ds(base, _CHUNK)], idx_v)
            pltpu.async_copy(table_hbm.at[idx_v], rows_v, sem).wait()

            def add_row(r, carry2):
                for j in range(vecs_per_row):
                    sl = pl.ds(j * _LANES, _LANES)
                    plsc.addupdate(rows_v.at[r, sl], pe_v[p + r, sl])
                return carry2

            lax.fori_loop(0, _CHUNK, add_row, 0, unroll=4)
            pltpu.sync_copy(rows_v, out_hbm.at[pl.ds(base, _CHUNK)])
            return carry

        lax.fori_loop(0, chunks_per_w, chunk_body, 0)

    return emb


def _pos_encoding(seq_len, dim):
    pos = jnp.arange(seq_len, dtype=jnp.float32)[:, None]
    div = jnp.exp(
        jnp.arange(0, dim, 2, dtype=jnp.float32) * (-np.log(10000.0) / dim))
    pe = jnp.zeros((seq_len, dim), dtype=jnp.float32)
    pe = pe.at[:, 0::2].set(jnp.sin(pos * div))
    pe = pe.at[:, 1::2].set(jnp.cos(pos * div))
    return pe


def kernel(input_ids, table):
    batch, seq_len = input_ids.shape
    _, dim = table.shape
    n_rows = batch * seq_len
    assert n_rows % (_NW * _CHUNK) == 0 and (n_rows // _NW) % _CHUNK == 0
    assert dim % _LANES == 0
    ids = input_ids.reshape(-1).astype(jnp.int32)
    pe2 = jnp.tile(_pos_encoding(seq_len, dim), (2, 1))
    out = _make_emb_kernel(n_rows, seq_len, dim)(ids, table.astype(jnp.float32), pe2)
    return out.reshape(batch, seq_len, dim)


# speedup vs baseline: 1.2361x; 1.2361x over previous
import functools

import jax
import jax.numpy as jnp
import numpy as np
from jax import lax
from jax.experimental import pallas as pl
from jax.experimental.pallas import tpu as pltpu
from jax.experimental.pallas import tpu_sc as plsc

_NUM_CORES = 2
_NUM_SUBCORES = 16
_LANES = 16
_NW = _NUM_CORES * _NUM_SUBCORES

_CHUNK = 128
_NBUF = 4
_AHEAD = 3


def _make_emb_kernel(n_rows, seq_len, dim):
    rows_per_w = n_rows // _NW
    chunks_per_w = rows_per_w // _CHUNK
    vecs_per_row = dim // _LANES
    n_groups = chunks_per_w // _NBUF
    assert chunks_per_w % _NBUF == 0 and chunks_per_w > _NBUF
    mesh = plsc.VectorSubcoreMesh(
        core_axis_name="c", subcore_axis_name="s",
        num_cores=_NUM_CORES, num_subcores=_NUM_SUBCORES)

    @functools.partial(
        pl.kernel,
        out_type=jax.ShapeDtypeStruct((n_rows, dim), jnp.float32),
        mesh=mesh,
        scratch_types=[
            pltpu.VMEM((chunks_per_w, _CHUNK), jnp.int32),
            pltpu.VMEM((_NBUF, _CHUNK, dim), jnp.float32),
            pltpu.VMEM((2 * seq_len, dim), jnp.float32),
            pltpu.SemaphoreType.DMA((_NBUF,)),
            pltpu.SemaphoreType.DMA((_NBUF,)),
        ],
        compiler_params=pltpu.CompilerParams(use_tc_tiling_on_sc=False),
    )
    def emb(ids_hbm, table_hbm, pe_hbm, out_hbm, idx_all, rows, pe_v, gsem, ssem):
        wid = lax.axis_index("c") * _NUM_SUBCORES + lax.axis_index("s")
        pltpu.sync_copy(pe_hbm, pe_v)
        pltpu.sync_copy(ids_hbm.at[pl.ds(wid * chunks_per_w, chunks_per_w)],
                        idx_all)
        wbase = wid * rows_per_w

        def gather_start(c, slot):
            pltpu.async_copy(table_hbm.at[idx_all.at[c]], rows.at[slot],
                             gsem.at[slot])

        def gather_wait(c, slot):
            pltpu.make_async_copy(table_hbm.at[idx_all.at[c]], rows.at[slot],
                                  gsem.at[slot]).wait()

        def store_start(c, slot):
            pltpu.async_copy(rows.at[slot],
                             out_hbm.at[pl.ds(wbase + c * _CHUNK, _CHUNK)],
                             ssem.at[slot])

        def store_wait(c, slot):
            pltpu.make_async_copy(rows.at[slot],
                                  out_hbm.at[pl.ds(wbase + c * _CHUNK, _CHUNK)],
                                  ssem.at[slot]).wait()

        for b in range(_AHEAD):
            gather_start(b, b)

        def group_body(g, carry):
            for b in range(_NBUF):
                c = g * _NBUF + b
                gather_wait(c, b)
                p = lax.rem(c * _CHUNK, seq_len)

                def add_row(r, carry2):
                    for j in range(vecs_per_row):
                        sl = pl.ds(j * _LANES, _LANES)
                        plsc.addupdate(rows.at[b, r, sl], pe_v[p + r, sl])
                    return carry2

                lax.fori_loop(0, _CHUNK, add_row, 0, unroll=4)

                f = c + _AHEAD
                bf = (b + _AHEAD) % _NBUF

                @pl.when(f < chunks_per_w)
                def _fire():
                    @pl.when(c >= 1)
                    def _drain():
                        store_wait(c - 1, bf)
                    gather_start(f, bf)

                store_start(c, b)
            return carry

        lax.fori_loop(0, n_groups, group_body, 0)
        for b in range(_NBUF):
            store_wait(chunks_per_w - _NBUF + b, b)

    return emb


def _pos_encoding(seq_len, dim):
    pos = jnp.arange(seq_len, dtype=jnp.float32)[:, None]
    div = jnp.exp(
        jnp.arange(0, dim, 2, dtype=jnp.float32) * (-np.log(10000.0) / dim))
    pe = jnp.zeros((seq_len, dim), dtype=jnp.float32)
    pe = pe.at[:, 0::2].set(jnp.sin(pos * div))
    pe = pe.at[:, 1::2].set(jnp.cos(pos * div))
    return pe


def kernel(input_ids, table):
    batch, seq_len = input_ids.shape
    _, dim = table.shape
    n_rows = batch * seq_len
    assert n_rows % (_NW * _CHUNK) == 0
    assert dim % _LANES == 0
    ids = input_ids.reshape(-1).astype(jnp.int32).reshape(-1, _CHUNK)
    pe2 = jnp.tile(_pos_encoding(seq_len, dim), (2, 1))
    out = _make_emb_kernel(n_rows, seq_len, dim)(ids, table.astype(jnp.float32), pe2)
    return out.reshape(batch, seq_len, dim)


# speedup vs baseline: 1.4880x; 1.2037x over previous
import functools

import jax
import jax.numpy as jnp
import numpy as np
from jax import lax
from jax.experimental import pallas as pl
from jax.experimental.pallas import tpu as pltpu
from jax.experimental.pallas import tpu_sc as plsc

_NUM_CORES = 2
_NUM_SUBCORES = 16
_LANES = 16
_NW = _NUM_CORES * _NUM_SUBCORES

_SPLIT = 2
_NBUF = 4
_AHEAD = 3


def _make_emb_kernel(batch, seq_len, dim):
    batches_per_w = batch // _NW
    chunk = seq_len // _SPLIT
    chunks_per_w = batches_per_w * _SPLIT
    vecs_per_row = dim // _LANES
    n_groups = chunks_per_w // _NBUF
    assert chunks_per_w % _NBUF == 0 and chunks_per_w > _NBUF
    assert _NBUF % _SPLIT == 0
    mesh = plsc.VectorSubcoreMesh(
        core_axis_name="c", subcore_axis_name="s",
        num_cores=_NUM_CORES, num_subcores=_NUM_SUBCORES)

    @functools.partial(
        pl.kernel,
        out_type=jax.ShapeDtypeStruct((batch, seq_len, dim), jnp.float32),
        mesh=mesh,
        scratch_types=[
            pltpu.VMEM((batches_per_w, _SPLIT, chunk), jnp.int32),
            pltpu.VMEM((_NBUF, chunk, dim), jnp.float32),
            pltpu.VMEM((seq_len, dim), jnp.float32),
            pltpu.SemaphoreType.DMA((_NBUF,)),
            pltpu.SemaphoreType.DMA((_NBUF,)),
        ],
        compiler_params=pltpu.CompilerParams(use_tc_tiling_on_sc=False),
    )
    def emb(ids_hbm, table_hbm, pe_hbm, out_hbm, idx_all, rows, pe_v, gsem, ssem):
        wid = lax.axis_index("c") * _NUM_SUBCORES + lax.axis_index("s")
        b0 = wid * batches_per_w
        pltpu.sync_copy(pe_hbm, pe_v)
        pltpu.sync_copy(ids_hbm.at[pl.ds(b0, batches_per_w)], idx_all)

        def gather_start(c, h, slot):
            pltpu.async_copy(table_hbm.at[idx_all.at[c // _SPLIT, h]],
                             rows.at[slot], gsem.at[slot])

        def gather_wait(c, h, slot):
            pltpu.make_async_copy(table_hbm.at[idx_all.at[c // _SPLIT, h]],
                                  rows.at[slot], gsem.at[slot]).wait()

        def store_start(c, h, slot):
            pltpu.async_copy(
                rows.at[slot],
                out_hbm.at[b0 + c // _SPLIT, pl.ds(h * chunk, chunk)],
                ssem.at[slot])

        def store_wait(c, h, slot):
            pltpu.make_async_copy(
                rows.at[slot],
                out_hbm.at[b0 + c // _SPLIT, pl.ds(h * chunk, chunk)],
                ssem.at[slot]).wait()

        for b in range(_AHEAD):
            gather_start(b, b % _SPLIT, b)

        def group_body(g, carry):
            for b in range(_NBUF):
                c = g * _NBUF + b
                h = b % _SPLIT
                gather_wait(c, h, b)

                def add_row(r, carry2):
                    for j in range(vecs_per_row):
                        sl = pl.ds(j * _LANES, _LANES)
                        plsc.addupdate(rows.at[b, r, sl],
                                       pe_v[h * chunk + r, sl])
                    return carry2

                lax.fori_loop(0, chunk, add_row, 0, unroll=4)

                f = c + _AHEAD
                bf = (b + _AHEAD) % _NBUF
                hf = f % _SPLIT

                @pl.when(f < chunks_per_w)
                def _fire():
                    @pl.when(c >= 1)
                    def _drain():
                        store_wait(c - 1, (h + 1) % _SPLIT, bf)
                    gather_start(f, hf, bf)

                store_start(c, h, b)
            return carry

        lax.fori_loop(0, n_groups, group_body, 0)
        for b in range(_NBUF):
            c = chunks_per_w - _NBUF + b
            store_wait(c, b % _SPLIT, b)

    return emb


def _pos_encoding(seq_len, dim):
    pos = jnp.arange(seq_len, dtype=jnp.float32)[:, None]
    div = jnp.exp(
        jnp.arange(0, dim, 2, dtype=jnp.float32) * (-np.log(10000.0) / dim))
    pe = jnp.zeros((seq_len, dim), dtype=jnp.float32)
    pe = pe.at[:, 0::2].set(jnp.sin(pos * div))
    pe = pe.at[:, 1::2].set(jnp.cos(pos * div))
    return pe


def kernel(input_ids, table):
    batch, seq_len = input_ids.shape
    _, dim = table.shape
    assert batch % _NW == 0 and seq_len % _SPLIT == 0
    assert (seq_len // _SPLIT) <= 128 and dim % _LANES == 0
    ids = input_ids.astype(jnp.int32).reshape(batch, _SPLIT, seq_len // _SPLIT)
    pe = _pos_encoding(seq_len, dim)
    return _make_emb_kernel(batch, seq_len, dim)(ids, table, pe)
